# baseline (speedup 1.0000x reference)
import jax
import jax.numpy as jnp
from jax import lax
from jax.experimental import pallas as pl
from jax.experimental.pallas import tpu as pltpu
from jax.experimental.pallas import tpu_sc as plsc

N = 10000
E = 320000
D = 128
H = 128
FD = 64
C = 40

NC = 2
NS = 16
NW = NC * NS
K = 128
EP = 10240
E_PAD = NW * EP
CH = EP // K
NPAD = 10240
NR = NPAD // NS


def _make_seg_sum(W):
  mesh = plsc.VectorSubcoreMesh(core_axis_name="c", subcore_axis_name="s")

  NB = 4
  KG = K // 2
  CG = EP // KG

  def body(x_hbm, src_hbm, dst_hbm, z_hbm, part_hbm, s0, s1, s2, s3,
           d0, d1, d2, d3, r0, r1, r2, r3, acc, m0, m1, m2, m3):
    srcv = (s0, s1, s2, s3)
    dstv = (d0, d1, d2, d3)
    rows = (r0, r1, r2, r3)
    sems = (m0, m1, m2, m3)
    cid = lax.axis_index("c")
    sid = lax.axis_index("s")
    wid = cid * NS + sid
    pltpu.sync_copy(z_hbm, rows[0])
    zbase = sid * NR
    for t in range(NR // KG):
      pltpu.sync_copy(rows[0], acc.at[pl.ds(zbase + t * KG, KG)])
    plsc.subcore_barrier()

    base = wid * EP

    def load_and_gather(g, l):
      off = base + g * KG
      pltpu.sync_copy(src_hbm.at[pl.ds(off, KG)], srcv[l])
      pltpu.sync_copy(dst_hbm.at[pl.ds(off, KG)], dstv[l])
      pltpu.async_copy(x_hbm.at[srcv[l]], rows[l], sems[l])

    def wait_and_scatter(g, l):
      del g
      pltpu.make_async_copy(x_hbm.at[srcv[l]], rows[l], sems[l]).wait()
      pltpu.sync_copy(rows[l], acc.at[dstv[l]], add=True)

    for l in range(NB):
      load_and_gather(l, l)

    def ring(j, carry):
      i = NB * j
      for l in range(NB):
        wait_and_scatter(i + l, l)
        load_and_gather(i + l + NB, l)
      return carry

    lax.fori_loop(0, CG // NB - 1, ring, 0)
    i = CG - NB
    for l in range(NB):
      wait_and_scatter(i + l, l)

    plsc.subcore_barrier()
    out_off = cid * NPAD + sid * NR
    for t in range(NR // KG):
      pltpu.sync_copy(acc.at[pl.ds(zbase + t * KG, KG)], rows[0])
      pltpu.sync_copy(rows[0], part_hbm.at[pl.ds(out_off + t * KG, KG)])

  return pl.kernel(
      body,
      out_type=jax.ShapeDtypeStruct((NC * NPAD, W), jnp.float32),
      mesh=mesh,
      scratch_types=(
          [pltpu.VMEM((K // 2,), jnp.int32) for _ in range(8)]
          + [pltpu.VMEM((K // 2, W), jnp.float32) for _ in range(4)]
          + [pltpu.VMEM_SHARED((NPAD, W), jnp.float32)]
          + [pltpu.SemaphoreType.DMA for _ in range(4)]
      ),
  )


_seg_sum = _make_seg_sum(D)


def _make_cnt():
  mesh = plsc.VectorSubcoreMesh(core_axis_name="c", subcore_axis_name="s")

  def body(dst_hbm, ones_hbm, z_hbm, cnt_hbm, dsts, buf, acc, sem):
    cid = lax.axis_index("c")
    sid = lax.axis_index("s")
    wid = cid * NS + sid
    pltpu.sync_copy(z_hbm, buf)
    zbase = sid * NR
    for t in range(NR // K):
      pltpu.sync_copy(buf, acc.at[pl.ds(zbase + t * K, K)])
    pltpu.sync_copy(ones_hbm, buf)
    pltpu.sync_copy(dst_hbm.at[wid], dsts)
    plsc.subcore_barrier()

    def chunk(g, carry):
      pltpu.sync_copy(buf, acc.at[dsts.at[g]], add=True)
      return carry

    lax.fori_loop(0, CH, chunk, 0)
    plsc.subcore_barrier()
    out_off = cid * NPAD + sid * NR
    for t in range(NR // K):
      pltpu.sync_copy(acc.at[pl.ds(zbase + t * K, K)], buf)
      pltpu.sync_copy(buf, cnt_hbm.at[pl.ds(out_off + t * K, K)])

  return pl.kernel(
      body,
      out_type=jax.ShapeDtypeStruct((NC * NPAD, D), jnp.float32),
      mesh=mesh,
      scratch_types=[
          pltpu.VMEM((CH, K), jnp.int32),
          pltpu.VMEM((K, D), jnp.float32),
          pltpu.VMEM_SHARED((NPAD, D), jnp.float32),
          pltpu.SemaphoreType.DMA,
      ],
  )


_cnt_sum = _make_cnt()

_DN = (((1,), (1,)), ((), ()))


def _bn_inline(x, g, b):
  mu = jnp.mean(x, axis=0, keepdims=True)
  var = jnp.mean((x - mu) ** 2, axis=0, keepdims=True)
  return g * (x - mu) * lax.rsqrt(var + 1e-5) + b


def _tc1_body(x_ref, part_ref, cnt_ref, wl_ref, bl_ref, wr_ref, g_ref, b_ref,
              out_ref):
  agg = part_ref[0:N, :] + part_ref[NPAD:NPAD + N, :]
  cnt = jnp.maximum(cnt_ref[0:N, 0:1] + cnt_ref[NPAD:NPAD + N, 0:1], 1.0)
  mean = agg / cnt
  pre = (lax.dot_general(mean, wl_ref[...], _DN,
                         preferred_element_type=jnp.float32)
         + bl_ref[...]
         + lax.dot_general(x_ref[...], wr_ref[...], _DN,
                           preferred_element_type=jnp.float32))
  out_ref[...] = jnp.maximum(_bn_inline(pre, g_ref[...], b_ref[...]), 0.0)


def _tc2_body(h_ref, part_ref, cnt_ref, xm_ref, wl1_ref, bl1_ref, wr1_ref,
              wm0_ref, bm0_ref, gm_ref, bm_ref, wm1_ref, bm1_ref,
              wc0_ref, bc0_ref, wc1_ref, bc1_ref, out_ref):
  f32 = jnp.float32
  cnt = jnp.maximum(cnt_ref[0:N, 0:1] + cnt_ref[NPAD:NPAD + N, 0:1], 1.0)
  agg = part_ref[0:N, :] + part_ref[NPAD:NPAD + N, :]
  mean = agg / cnt
  h = h_ref[...]
  h_gsage = (lax.dot_general(mean, wl1_ref[...], _DN,
                             preferred_element_type=f32)
             + bl1_ref[...]
             + lax.dot_general(h, wr1_ref[...], _DN,
                               preferred_element_type=f32))
  m = lax.dot_general(xm_ref[...], wm0_ref[...], _DN,
                      preferred_element_type=f32) + bm0_ref[...]
  m = jax.nn.sigmoid(_bn_inline(m, gm_ref[...], bm_ref[...]))
  h_mlp = lax.dot_general(m, wm1_ref[...], _DN,
                          preferred_element_type=f32) + bm1_ref[...]
  wc0 = wc0_ref[...]
  z = jnp.maximum(
      lax.dot_general(h_gsage, wc0[:, 0:3 * FD], _DN,
                      preferred_element_type=f32)
      + lax.dot_general(h_mlp, wc0[:, 3 * FD:4 * FD], _DN,
                        preferred_element_type=f32)
      + bc0_ref[...], 0.0)
  out_ref[...] = lax.dot_general(z, wc1_ref[...], _DN,
                                 preferred_element_type=f32) + bc1_ref[...]


_tc1 = pl.pallas_call(
    _tc1_body, out_shape=jax.ShapeDtypeStruct((N, H), jnp.float32))
_tc2 = pl.pallas_call(
    _tc2_body, out_shape=jax.ShapeDtypeStruct((N, C), jnp.float32))


def kernel(x_gsage, x_mlp, edge_index, Wl0, bl0, Wr0, g_g, b_g, Wl1, bl1,
           Wr1, Wm0, bm0, g_m, b_m, Wm1, bm1, Wc0, bc0, Wc1, bc1):
  src = edge_index[0]
  dst = edge_index[1]
  pad = E_PAD - E
  ar = jnp.arange(pad, dtype=jnp.int32)
  srcp = jnp.concatenate([src, ar % N])
  dstp = jnp.concatenate([dst, N + ar % (NPAD - N)])
  dstp_c = dstp.reshape(NW, CH, K)
  ones = jnp.ones((K, D), jnp.float32)
  z_c = jnp.zeros((K, D), jnp.float32)
  z_g = jnp.zeros((K // 2, D), jnp.float32)
  r = lambda v: v.reshape(1, -1)

  cntp = _cnt_sum(dstp_c, ones, z_c)
  part0 = _seg_sum(x_gsage, srcp, dstp, z_g)
  h = _tc1(x_gsage, part0, cntp, Wl0, r(bl0), Wr0, r(g_g), r(b_g))
  part1 = _seg_sum(h, srcp, dstp, z_g)
  out = _tc2(h, part1, cntp, x_mlp, Wl1, r(bl1), Wr1, Wm0, r(bm0), r(g_m),
             r(b_m), Wm1, r(bm1), Wc0, r(bc0), Wc1, r(bc1))
  return out


# speedup vs baseline: 1.1961x; 1.1961x over previous
import jax
import jax.numpy as jnp
from jax import lax
from jax.experimental import pallas as pl
from jax.experimental.pallas import tpu as pltpu
from jax.experimental.pallas import tpu_sc as plsc

N = 10000
E = 320000
D = 128
H = 128
FD = 64
C = 40

NC = 2
NS = 16
NW = NC * NS
K = 128
EP = 10240
E_PAD = NW * EP
CH = EP // K
NPAD = 10240
NR = NPAD // NS


def _make_seg_sum(W):
  mesh = plsc.VectorSubcoreMesh(core_axis_name="c", subcore_axis_name="s")

  def body(x_hbm, src_hbm, dst_hbm, z_hbm, part_hbm, dsts, srcv0, srcv1,
           rows0, rows1, acc, sem0, sem1):
    cid = lax.axis_index("c")
    sid = lax.axis_index("s")
    wid = cid * NS + sid
    pltpu.sync_copy(z_hbm, rows0)
    zbase = sid * NR
    for t in range(NR // K):
      pltpu.sync_copy(rows0, acc.at[pl.ds(zbase + t * K, K)])
    pltpu.sync_copy(dst_hbm.at[wid], dsts)
    plsc.subcore_barrier()

    base = wid * EP

    pltpu.sync_copy(src_hbm.at[pl.ds(base, K)], srcv0)
    pltpu.async_copy(x_hbm.at[srcv0], rows0, sem0)

    def pair(j, carry):
      i0 = 2 * j
      pltpu.make_async_copy(x_hbm.at[srcv0], rows0, sem0).wait()
      pltpu.sync_copy(src_hbm.at[pl.ds(base + (i0 + 1) * K, K)], srcv1)
      pltpu.async_copy(x_hbm.at[srcv1], rows1, sem1)
      pltpu.sync_copy(rows0, acc.at[dsts.at[i0]], add=True)
      pltpu.make_async_copy(x_hbm.at[srcv1], rows1, sem1).wait()
      pltpu.sync_copy(src_hbm.at[pl.ds(base + (i0 + 2) * K, K)], srcv0)
      pltpu.async_copy(x_hbm.at[srcv0], rows0, sem0)
      pltpu.sync_copy(rows1, acc.at[dsts.at[i0 + 1]], add=True)
      return carry

    lax.fori_loop(0, CH // 2 - 1, pair, 0)
    i = CH - 2
    pltpu.make_async_copy(x_hbm.at[srcv0], rows0, sem0).wait()
    pltpu.sync_copy(src_hbm.at[pl.ds(base + (i + 1) * K, K)], srcv1)
    pltpu.async_copy(x_hbm.at[srcv1], rows1, sem1)
    pltpu.sync_copy(rows0, acc.at[dsts.at[i]], add=True)
    pltpu.make_async_copy(x_hbm.at[srcv1], rows1, sem1).wait()
    pltpu.sync_copy(rows1, acc.at[dsts.at[i + 1]], add=True)

    plsc.subcore_barrier()
    out_off = cid * NPAD + sid * NR
    for t in range(NR // K):
      pltpu.sync_copy(acc.at[pl.ds(zbase + t * K, K)], rows0)
      pltpu.sync_copy(rows0, part_hbm.at[pl.ds(out_off + t * K, K)])

  return pl.kernel(
      body,
      out_type=jax.ShapeDtypeStruct((NC * NPAD, W), jnp.float32),
      mesh=mesh,
      scratch_types=[
          pltpu.VMEM((CH, K), jnp.int32),
          pltpu.VMEM((K,), jnp.int32),
          pltpu.VMEM((K,), jnp.int32),
          pltpu.VMEM((K, W), jnp.float32),
          pltpu.VMEM((K, W), jnp.float32),
          pltpu.VMEM_SHARED((NPAD, W), jnp.float32),
          pltpu.SemaphoreType.DMA,
          pltpu.SemaphoreType.DMA,
      ],
  )


_seg_sum = _make_seg_sum(D)


def _make_cnt():
  mesh = plsc.VectorSubcoreMesh(core_axis_name="c", subcore_axis_name="s")

  def body(dst_hbm, ones_hbm, z_hbm, cnt_hbm, dsts, buf, acc, sem):
    cid = lax.axis_index("c")
    sid = lax.axis_index("s")
    wid = cid * NS + sid
    pltpu.sync_copy(z_hbm, buf)
    zbase = sid * NR
    for t in range(NR // K):
      pltpu.sync_copy(buf, acc.at[pl.ds(zbase + t * K, K)])
    pltpu.sync_copy(ones_hbm, buf)
    pltpu.sync_copy(dst_hbm.at[wid], dsts)
    plsc.subcore_barrier()

    def chunk(g, carry):
      pltpu.sync_copy(buf, acc.at[dsts.at[g]], add=True)
      return carry

    lax.fori_loop(0, CH, chunk, 0)
    plsc.subcore_barrier()
    out_off = cid * NPAD + sid * NR
    for t in range(NR // K):
      pltpu.sync_copy(acc.at[pl.ds(zbase + t * K, K)], buf)
      pltpu.sync_copy(buf, cnt_hbm.at[pl.ds(out_off + t * K, K)])

  return pl.kernel(
      body,
      out_type=jax.ShapeDtypeStruct((NC * NPAD, D), jnp.float32),
      mesh=mesh,
      scratch_types=[
          pltpu.VMEM((CH, K), jnp.int32),
          pltpu.VMEM((K, D), jnp.float32),
          pltpu.VMEM_SHARED((NPAD, D), jnp.float32),
          pltpu.SemaphoreType.DMA,
      ],
  )


_cnt_sum = _make_cnt()

_DN = (((1,), (1,)), ((), ()))


def _bn_inline(x, g, b):
  mu = jnp.mean(x, axis=0, keepdims=True)
  var = jnp.mean((x - mu) ** 2, axis=0, keepdims=True)
  return g * (x - mu) * lax.rsqrt(var + 1e-5) + b


def _tc1_body(x_ref, part_ref, cnt_ref, wl_ref, bl_ref, wr_ref, g_ref, b_ref,
              out_ref):
  agg = part_ref[0:N, :] + part_ref[NPAD:NPAD + N, :]
  cnt = jnp.maximum(cnt_ref[0:N, 0:1] + cnt_ref[NPAD:NPAD + N, 0:1], 1.0)
  mean = agg / cnt
  pre = (lax.dot_general(mean, wl_ref[...], _DN,
                         preferred_element_type=jnp.float32)
         + bl_ref[...]
         + lax.dot_general(x_ref[...], wr_ref[...], _DN,
                           preferred_element_type=jnp.float32))
  out_ref[...] = jnp.maximum(_bn_inline(pre, g_ref[...], b_ref[...]), 0.0)


def _tc2_body(h_ref, part_ref, cnt_ref, xm_ref, wl1_ref, bl1_ref, wr1_ref,
              wm0_ref, bm0_ref, gm_ref, bm_ref, wm1_ref, bm1_ref,
              wc0_ref, bc0_ref, wc1_ref, bc1_ref, out_ref):
  f32 = jnp.float32
  cnt = jnp.maximum(cnt_ref[0:N, 0:1] + cnt_ref[NPAD:NPAD + N, 0:1], 1.0)
  agg = part_ref[0:N, :] + part_ref[NPAD:NPAD + N, :]
  mean = agg / cnt
  h = h_ref[...]
  h_gsage = (lax.dot_general(mean, wl1_ref[...], _DN,
                             preferred_element_type=f32)
             + bl1_ref[...]
             + lax.dot_general(h, wr1_ref[...], _DN,
                               preferred_element_type=f32))
  m = lax.dot_general(xm_ref[...], wm0_ref[...], _DN,
                      preferred_element_type=f32) + bm0_ref[...]
  m = jax.nn.sigmoid(_bn_inline(m, gm_ref[...], bm_ref[...]))
  h_mlp = lax.dot_general(m, wm1_ref[...], _DN,
                          preferred_element_type=f32) + bm1_ref[...]
  wc0 = wc0_ref[...]
  z = jnp.maximum(
      lax.dot_general(h_gsage, wc0[:, 0:3 * FD], _DN,
                      preferred_element_type=f32)
      + lax.dot_general(h_mlp, wc0[:, 3 * FD:4 * FD], _DN,
                        preferred_element_type=f32)
      + bc0_ref[...], 0.0)
  out_ref[...] = lax.dot_general(z, wc1_ref[...], _DN,
                                 preferred_element_type=f32) + bc1_ref[...]


_tc1 = pl.pallas_call(
    _tc1_body, out_shape=jax.ShapeDtypeStruct((N, H), jnp.float32))
_tc2 = pl.pallas_call(
    _tc2_body, out_shape=jax.ShapeDtypeStruct((N, C), jnp.float32))


def kernel(x_gsage, x_mlp, edge_index, Wl0, bl0, Wr0, g_g, b_g, Wl1, bl1,
           Wr1, Wm0, bm0, g_m, b_m, Wm1, bm1, Wc0, bc0, Wc1, bc1):
  src = edge_index[0]
  dst = edge_index[1]
  pad = E_PAD - E
  ar = jnp.arange(pad, dtype=jnp.int32)
  srcp = jnp.concatenate([src, ar % N])
  dstp = jnp.concatenate([dst, N + ar % (NPAD - N)]).reshape(NW, CH, K)
  ones = jnp.ones((K, D), jnp.float32)
  z = jnp.zeros((K, D), jnp.float32)
  r = lambda v: v.reshape(1, -1)

  cntp = _cnt_sum(dstp, ones, z)
  part0 = _seg_sum(x_gsage, srcp, dstp, z)
  h = _tc1(x_gsage, part0, cntp, Wl0, r(bl0), Wr0, r(g_g), r(b_g))
  part1 = _seg_sum(h, srcp, dstp, z)
  out = _tc2(h, part1, cntp, x_mlp, Wl1, r(bl1), Wr1, Wm0, r(bm0), r(g_m),
             r(b_m), Wm1, r(bm1), Wc0, r(bc0), Wc1, r(bc1))
  return out
